# Initial kernel scaffold; baseline (speedup 1.0000x reference)
import jax
import jax.numpy as jnp
from jax.experimental import pallas as pl


def kernel(x, edge_index, W0, b0, W1, b1, g0wih, g0whh, g0bih, g0bhh, g1wih, g1whh, g1bih, g1bhh, p0, p1, ln_g, ln_b):
    raise NotImplementedError("write your pallas kernel here")


# speedup vs baseline: 21.6677x; 21.6677x over previous
import functools

import jax
import jax.numpy as jnp
from jax import lax
from jax.experimental import pallas as pl
from jax.experimental.pallas import tpu as pltpu
from jax.experimental.pallas import tpu_sc as plsc

N = 10000
D = 128
HID = 16
OUT = 16
TK = 8
E = 320000
H0 = HID * D + HID
H1 = OUT * HID + OUT
NC, NS, CH = 2, 16, 128
NCH = 80
EPAD = NC * NS * NCH * CH
NPAD = 10240
SLAB = NPAD // NS
GB = 16


def _sc_mesh():
    return plsc.VectorSubcoreMesh(core_axis_name="c", subcore_axis_name="s")


_SC_PARAMS = pltpu.CompilerParams(use_tc_tiling_on_sc=False)


def _deg_body(dst_hbm, zeros_hbm, out_hbm, idx_v, ones_v, deg_sh, sem):
    c = lax.axis_index("c")
    s = lax.axis_index("s")
    for i in range(CH // 16):
        ones_v[pl.ds(i * 16, 16)] = jnp.ones((16,), jnp.float32)
    pltpu.sync_copy(zeros_hbm.at[pl.ds(s * SLAB, SLAB)],
                    deg_sh.at[pl.ds(s * SLAB, SLAB)])
    pltpu.sync_copy(dst_hbm.at[c, s], idx_v)
    plsc.subcore_barrier()

    def group(g, carry):
        cps = [pltpu.async_copy(ones_v, deg_sh.at[idx_v.at[g * GB + b]], sem,
                                add=True)
               for b in range(GB)]
        for cp in cps:
            cp.wait()
        return carry

    lax.fori_loop(0, NCH // GB, group, 0)
    plsc.subcore_barrier()
    pltpu.sync_copy(deg_sh.at[pl.ds(s * SLAB, SLAB)],
                    out_hbm.at[c, pl.ds(s * SLAB, SLAB)])


def _sc_deg(dst3, zeros1):
    return pl.kernel(
        _deg_body,
        out_type=jax.ShapeDtypeStruct((NC, NPAD), jnp.float32),
        mesh=_sc_mesh(),
        compiler_params=_SC_PARAMS,
        scratch_types=[
            pltpu.VMEM((NCH, CH), jnp.int32),
            pltpu.VMEM((CH,), jnp.float32),
            pltpu.VMEM_SHARED((NPAD,), jnp.float32),
            pltpu.SemaphoreType.DMA,
        ],
    )(dst3, zeros1)


def _agg_body(yw_hbm, src_hbm, dst_hbm, zeros_hbm, out_hbm,
              src_v, dst_v, rows_v, acc_sh, gsem, ssem):
    c = lax.axis_index("c")
    s = lax.axis_index("s")
    pltpu.sync_copy(zeros_hbm.at[pl.ds(s * SLAB, SLAB)],
                    acc_sh.at[pl.ds(s * SLAB, SLAB)])
    pltpu.sync_copy(src_hbm.at[c, s], src_v)
    pltpu.sync_copy(dst_hbm.at[c, s], dst_v)
    plsc.subcore_barrier()

    def group(g, carry):
        gcps = [pltpu.async_copy(yw_hbm.at[src_v.at[g * GB + b]],
                                 rows_v.at[pl.ds(b * CH, CH)], gsem)
                for b in range(GB)]
        for cp in gcps:
            cp.wait()
        scps = [pltpu.async_copy(rows_v.at[pl.ds(b * CH, CH)],
                                 acc_sh.at[dst_v.at[g * GB + b]], ssem,
                                 add=True)
                for b in range(GB)]
        for cp in scps:
            cp.wait()
        return carry

    lax.fori_loop(0, NCH // GB, group, 0)
    plsc.subcore_barrier()
    pltpu.sync_copy(acc_sh.at[pl.ds(s * SLAB, SLAB)],
                    out_hbm.at[c, pl.ds(s * SLAB, SLAB)])


def _sc_agg(yw, src3, dst3, zeros2):
    return pl.kernel(
        _agg_body,
        out_type=jax.ShapeDtypeStruct((NC, NPAD, HID), jnp.float32),
        mesh=_sc_mesh(),
        compiler_params=_SC_PARAMS,
        scratch_types=[
            pltpu.VMEM((NCH, CH), jnp.int32),
            pltpu.VMEM((NCH, CH), jnp.int32),
            pltpu.VMEM((GB * CH, HID), jnp.float32),
            pltpu.VMEM_SHARED((NPAD, HID), jnp.float32),
            pltpu.SemaphoreType.DMA,
            pltpu.SemaphoreType.DMA,
        ],
    )(yw, src3, dst3, zeros2)


def _front_body(h_ref, p_ref, z_ref, *, nvalid, k, d):
    p = p_ref[0, :]
    pn = p / (jnp.sqrt(jnp.sum(p * p)) + 1e-8)
    y = jnp.dot(h_ref[...], pn[:, None], preferred_element_type=jnp.float32)
    rows = lax.broadcasted_iota(jnp.int32, y.shape, 0)
    y = jnp.where(rows < nvalid, y, -jnp.inf)
    for j in range(k):
        m = jnp.max(y)
        idx = jnp.min(jnp.where(y == m, rows, nvalid))
        w = jnp.tanh(m)
        row = h_ref[pl.ds(idx, 1), :]
        z_ref[0, pl.ds(j * d, d)] = row[0, :] * w
        y = jnp.where(rows == idx, -jnp.inf, y)


def _front(h, p2, nvalid, k, d):
    return pl.pallas_call(
        functools.partial(_front_body, nvalid=nvalid, k=k, d=d),
        out_shape=jax.ShapeDtypeStruct((1, k * d), jnp.float32),
    )(h, p2)


def _mv2_body(wih_ref, whh_ref, zin_ref, hin_ref, gi_ref, gh_ref):
    gi_ref[...] = jnp.dot(wih_ref[...], zin_ref[...],
                          preferred_element_type=jnp.float32)
    gh_ref[...] = jnp.dot(whh_ref[...], hin_ref[...],
                          preferred_element_type=jnp.float32)


def _gru_mv(wih, whh, zin, hin, br):
    r, ki = wih.shape
    kh = whh.shape[1]
    return pl.pallas_call(
        _mv2_body,
        grid=(r // br,),
        in_specs=[
            pl.BlockSpec((br, ki), lambda i: (i, 0)),
            pl.BlockSpec((br, kh), lambda i: (i, 0)),
            pl.BlockSpec((ki, 8), lambda i: (0, 0)),
            pl.BlockSpec((kh, 8), lambda i: (0, 0)),
        ],
        out_specs=[
            pl.BlockSpec((br, 8), lambda i: (i, 0)),
            pl.BlockSpec((br, 8), lambda i: (i, 0)),
        ],
        out_shape=[
            jax.ShapeDtypeStruct((r, 8), jnp.float32),
            jax.ShapeDtypeStruct((r, 8), jnp.float32),
        ],
    )(wih, whh, zin, hin)


def _gate_body(gi_ref, gh_ref, bih_ref, bhh_ref, h0_ref, v_ref):
    r = jax.nn.sigmoid(gi_ref[0, :] + bih_ref[0, :]
                       + gh_ref[0, :] + bhh_ref[0, :])
    z = jax.nn.sigmoid(gi_ref[1, :] + bih_ref[1, :]
                       + gh_ref[1, :] + bhh_ref[1, :])
    n = jnp.tanh(gi_ref[2, :] + bih_ref[2, :]
                 + r * (gh_ref[2, :] + bhh_ref[2, :]))
    v_ref[0, :] = (1.0 - z) * n + z * h0_ref[0, :]


def _gate(gi3, gh3, bih3, bhh3, h02):
    h = gi3.shape[1]
    return pl.pallas_call(
        _gate_body,
        out_shape=jax.ShapeDtypeStruct((1, h), jnp.float32),
    )(gi3, gh3, bih3, bhh3, h02)


def _xw_body(x_ref, wnt_ref, deg_ref, yw_ref):
    dinv = lax.rsqrt(deg_ref[0, :] + deg_ref[1, :] + 1.0)
    yw_ref[...] = dinv[:, None] * jnp.dot(x_ref[...], wnt_ref[...],
                                          preferred_element_type=jnp.float32)


def _xw(xp, wnt, deg, bn_rows):
    d = xp.shape[1]
    h = wnt.shape[1]
    return pl.pallas_call(
        _xw_body,
        grid=(NPAD // bn_rows,),
        in_specs=[
            pl.BlockSpec((bn_rows, d), lambda i: (i, 0)),
            pl.BlockSpec((d, h), lambda i: (0, 0)),
            pl.BlockSpec((2, bn_rows), lambda i: (0, i)),
        ],
        out_specs=pl.BlockSpec((bn_rows, h), lambda i: (i, 0)),
        out_shape=jax.ShapeDtypeStruct((NPAD, h), jnp.float32),
    )(xp, wnt, deg)


def _post_body(agg_ref, yw_ref, deg_ref, bn_ref, g_ref, b_ref, out_ref, *,
               do_ln):
    dinv = lax.rsqrt(deg_ref[0, :] + deg_ref[1, :] + 1.0)
    o = dinv[:, None] * (agg_ref[0] + agg_ref[1] + yw_ref[...]) \
        + bn_ref[0, :][None, :]
    if do_ln:
        mu = jnp.mean(o, axis=-1, keepdims=True)
        var = jnp.mean((o - mu) ** 2, axis=-1, keepdims=True)
        o = (o - mu) * lax.rsqrt(var + 1e-5) * g_ref[0, :][None, :] \
            + b_ref[0, :][None, :]
        o = jnp.maximum(o, 0.0)
    out_ref[...] = o


def _post(agg, yw, deg, bn2, g2, b2, do_ln, bn_rows):
    h = yw.shape[1]
    return pl.pallas_call(
        functools.partial(_post_body, do_ln=do_ln),
        grid=(NPAD // bn_rows,),
        in_specs=[
            pl.BlockSpec((2, bn_rows, h), lambda i: (0, i, 0)),
            pl.BlockSpec((bn_rows, h), lambda i: (i, 0)),
            pl.BlockSpec((2, bn_rows), lambda i: (0, i)),
            pl.BlockSpec((1, h), lambda i: (0, 0)),
            pl.BlockSpec((1, h), lambda i: (0, 0)),
            pl.BlockSpec((1, h), lambda i: (0, 0)),
        ],
        out_specs=pl.BlockSpec((bn_rows, h), lambda i: (i, 0)),
        out_shape=jax.ShapeDtypeStruct((NPAD, h), jnp.float32),
    )(agg, yw, deg, bn2, g2, b2)


def kernel(x, edge_index, W0, b0, W1, b1, g0wih, g0whh, g0bih, g0bhh,
           g1wih, g1whh, g1bih, g1bhh, p0, p1, ln_g, ln_b):
    src = edge_index[0]
    dst = edge_index[1]
    padi = jnp.full((EPAD - E,), N, jnp.int32)
    src3 = jnp.concatenate([src, padi]).reshape(NC, NS, NCH, CH)
    dst3 = jnp.concatenate([dst, padi]).reshape(NC, NS, NCH, CH)
    zeros1 = jnp.zeros((NPAD,), jnp.float32)
    zeros2 = jnp.zeros((NPAD, HID), jnp.float32)

    deg = _sc_deg(dst3, zeros1)
    xpad = jnp.concatenate(
        [x, jnp.zeros((NPAD - N, D), jnp.float32)], axis=0)

    Z0 = _front(x, p0.reshape(1, D), N, TK, D)
    hid0 = jnp.concatenate([W0.reshape(-1), b0])
    zin0 = jnp.broadcast_to(Z0.reshape(-1)[:, None], (TK * D, 8))
    hin0 = jnp.broadcast_to(hid0[:, None], (H0, 8))
    gi0, gh0 = _gru_mv(g0wih, g0whh, zin0, hin0, 1032)
    v0 = _gate(gi0[:, 0].reshape(3, H0), gh0[:, 0].reshape(3, H0),
               g0bih.reshape(3, H0), g0bhh.reshape(3, H0),
               hid0.reshape(1, H0))
    wnt0 = v0[0, :HID * D].reshape(HID, D).T
    bn0 = v0[0, HID * D:].reshape(1, HID)
    yw0 = _xw(xpad, wnt0, deg, 2048)
    agg0 = _sc_agg(yw0, src3, dst3, zeros2)
    h1 = _post(agg0, yw0, deg, bn0, ln_g.reshape(1, HID),
               ln_b.reshape(1, HID), True, 2048)

    Z1 = _front(h1, p1.reshape(1, HID), N, TK, HID)
    hid1 = jnp.concatenate([W1.reshape(-1), b1])
    zin1 = jnp.broadcast_to(Z1.reshape(-1)[:, None], (TK * HID, 8))
    hin1 = jnp.broadcast_to(hid1[:, None], (H1, 8))
    gi1, gh1 = _gru_mv(g1wih, g1whh, zin1, hin1, 816)
    v1 = _gate(gi1[:, 0].reshape(3, H1), gh1[:, 0].reshape(3, H1),
               g1bih.reshape(3, H1), g1bhh.reshape(3, H1),
               hid1.reshape(1, H1))
    wnt1 = v1[0, :OUT * HID].reshape(OUT, HID).T
    bn1 = v1[0, OUT * HID:].reshape(1, OUT)
    yw1 = _xw(h1, wnt1, deg, 2048)
    agg1 = _sc_agg(yw1, src3, dst3, zeros2)
    h2 = _post(agg1, yw1, deg, bn1, ln_g.reshape(1, HID),
               ln_b.reshape(1, HID), False, 2048)

    return h2[:N]


# speedup vs baseline: 23.8599x; 1.1012x over previous
import functools

import jax
import jax.numpy as jnp
from jax import lax
from jax.experimental import pallas as pl
from jax.experimental.pallas import tpu as pltpu
from jax.experimental.pallas import tpu_sc as plsc

N = 10000
D = 128
HID = 16
OUT = 16
TK = 8
E = 320000
H0 = HID * D + HID
H1 = OUT * HID + OUT
NC, NS, CH = 2, 16, 128
NCH = 80
EPAD = NC * NS * NCH * CH
NPAD = 10240
SLAB = NPAD // NS
GB = 8


def _sc_mesh():
    return plsc.VectorSubcoreMesh(core_axis_name="c", subcore_axis_name="s")


_SC_PARAMS = pltpu.CompilerParams(use_tc_tiling_on_sc=False)


def _deg_body(dst_hbm, zeros_hbm, out_hbm, idx_v, ones_v, deg_sh, sem):
    c = lax.axis_index("c")
    s = lax.axis_index("s")
    for j in range(CH // 16):
        ones_v[pl.ds(j * 16, 16)] = jnp.ones((16,), jnp.float32)
    pltpu.sync_copy(zeros_hbm.at[pl.ds(s * SLAB, SLAB)],
                    deg_sh.at[pl.ds(s * SLAB, SLAB)])
    pltpu.sync_copy(dst_hbm.at[c, s], idx_v)
    plsc.subcore_barrier()
    cps = [pltpu.async_copy(ones_v, deg_sh.at[idx_v.at[j]], sem, add=True)
           for j in range(NCH)]
    for cp in cps:
        cp.wait()
    plsc.subcore_barrier()
    pltpu.sync_copy(deg_sh.at[pl.ds(s * SLAB, SLAB)],
                    out_hbm.at[c, pl.ds(s * SLAB, SLAB)])


def _sc_deg(dst3, zeros1):
    return pl.kernel(
        _deg_body,
        out_type=jax.ShapeDtypeStruct((NC, NPAD), jnp.float32),
        mesh=_sc_mesh(),
        compiler_params=_SC_PARAMS,
        scratch_types=[
            pltpu.VMEM((NCH, CH), jnp.int32),
            pltpu.VMEM((CH,), jnp.float32),
            pltpu.VMEM_SHARED((NPAD,), jnp.float32),
            pltpu.SemaphoreType.DMA,
        ],
    )(dst3, zeros1)


def _agg_body(yw_hbm, src_hbm, dst_hbm, zeros_hbm, out_hbm,
              src_v, dst_v, r_a, r_b, acc_sh, gs_a, gs_b, ss_a, ss_b):
    c = lax.axis_index("c")
    s = lax.axis_index("s")
    pltpu.sync_copy(zeros_hbm.at[pl.ds(s * SLAB, SLAB)],
                    acc_sh.at[pl.ds(s * SLAB, SLAB)])
    pltpu.sync_copy(src_hbm.at[c, s], src_v)
    pltpu.sync_copy(dst_hbm.at[c, s], dst_v)
    plsc.subcore_barrier()

    NG = NCH // GB
    bufs = [(r_a, gs_a, ss_a), (r_b, gs_b, ss_b)]
    gcp, scp = {}, {}

    def fire_g(g):
        r, gs, _ = bufs[g % 2]
        gcp[g] = [pltpu.async_copy(yw_hbm.at[src_v.at[g * GB + b]],
                                   r.at[pl.ds(b * CH, CH)], gs)
                  for b in range(GB)]

    def fire_s(g):
        r, _, ss = bufs[g % 2]
        scp[g] = [pltpu.async_copy(r.at[pl.ds(b * CH, CH)],
                                   acc_sh.at[dst_v.at[g * GB + b]], ss,
                                   add=True)
                  for b in range(GB)]

    def drain(cps):
        for cp in cps:
            cp.wait()

    fire_g(0)
    for g in range(NG):
        if g + 1 < NG:
            if g >= 1:
                drain(scp[g - 1])
            fire_g(g + 1)
        drain(gcp[g])
        fire_s(g)
    drain(scp[NG - 2])
    drain(scp[NG - 1])
    plsc.subcore_barrier()
    pltpu.sync_copy(acc_sh.at[pl.ds(s * SLAB, SLAB)],
                    out_hbm.at[c, pl.ds(s * SLAB, SLAB)])


def _sc_agg(yw, src3, dst3, zeros2):
    return pl.kernel(
        _agg_body,
        out_type=jax.ShapeDtypeStruct((NC, NPAD, HID), jnp.float32),
        mesh=_sc_mesh(),
        compiler_params=_SC_PARAMS,
        scratch_types=[
            pltpu.VMEM((NCH, CH), jnp.int32),
            pltpu.VMEM((NCH, CH), jnp.int32),
            pltpu.VMEM((GB * CH, HID), jnp.float32),
            pltpu.VMEM((GB * CH, HID), jnp.float32),
            pltpu.VMEM_SHARED((NPAD, HID), jnp.float32),
            pltpu.SemaphoreType.DMA,
            pltpu.SemaphoreType.DMA,
            pltpu.SemaphoreType.DMA,
            pltpu.SemaphoreType.DMA,
        ],
    )(yw, src3, dst3, zeros2)


def _front_body(h_ref, p_ref, z_ref, *, nvalid, k, d, m):
    p = p_ref[0, :]
    pn = p / (jnp.sqrt(jnp.sum(p * p)) + 1e-8)
    y = jnp.dot(h_ref[...], pn[:, None], preferred_element_type=jnp.float32)
    y2 = y.reshape(m // 128, 128)
    gidx = lax.broadcasted_iota(jnp.int32, y2.shape, 0) * 128 \
        + lax.broadcasted_iota(jnp.int32, y2.shape, 1)
    y2 = jnp.where(gidx < nvalid, y2, -jnp.inf)
    for j in range(k):
        mx = jnp.max(y2)
        idx = jnp.min(jnp.where(y2 == mx, gidx, nvalid))
        w = jnp.tanh(mx)
        row = h_ref[pl.ds(idx, 1), :]
        z_ref[0, pl.ds(j * d, d)] = row[0, :] * w
        y2 = jnp.where(gidx == idx, -jnp.inf, y2)


def _front(h, p2, nvalid, k, d):
    m = h.shape[0]
    return pl.pallas_call(
        functools.partial(_front_body, nvalid=nvalid, k=k, d=d, m=m),
        out_shape=jax.ShapeDtypeStruct((1, k * d), jnp.float32),
    )(h, p2)


def _gru_body(wih_ref, whh_ref, zc_ref, hc_ref, bih_ref, bhh_ref, h0_ref,
              v_ref, gi_s, gh_s, *, rb, nsteps, h):
    i = pl.program_id(0)
    gi = jnp.dot(wih_ref[...], zc_ref[...],
                 preferred_element_type=jnp.float32)
    gh = jnp.dot(whh_ref[...], hc_ref[...],
                 preferred_element_type=jnp.float32)
    gi_s[pl.ds(i * rb, rb), :] = gi
    gh_s[pl.ds(i * rb, rb), :] = gh

    @pl.when(i == nsteps - 1)
    def _():
        r = jax.nn.sigmoid(gi_s[pl.ds(0, h), :] + bih_ref[pl.ds(0, h), :]
                           + gh_s[pl.ds(0, h), :] + bhh_ref[pl.ds(0, h), :])
        z = jax.nn.sigmoid(
            gi_s[pl.ds(h, h), :] + bih_ref[pl.ds(h, h), :]
            + gh_s[pl.ds(h, h), :] + bhh_ref[pl.ds(h, h), :])
        n = jnp.tanh(
            gi_s[pl.ds(2 * h, h), :] + bih_ref[pl.ds(2 * h, h), :]
            + r * (gh_s[pl.ds(2 * h, h), :] + bhh_ref[pl.ds(2 * h, h), :]))
        v_ref[...] = (1.0 - z) * n + z * h0_ref[...]


def _gru(wih, whh, zcol, hcol, bihc, bhhc, h0c, rb):
    r3, ki = wih.shape
    kh = whh.shape[1]
    h = r3 // 3
    nsteps = r3 // rb
    return pl.pallas_call(
        functools.partial(_gru_body, rb=rb, nsteps=nsteps, h=h),
        grid=(nsteps,),
        in_specs=[
            pl.BlockSpec((rb, ki), lambda i: (i, 0)),
            pl.BlockSpec((rb, kh), lambda i: (i, 0)),
            pl.BlockSpec((ki, 1), lambda i: (0, 0)),
            pl.BlockSpec((kh, 1), lambda i: (0, 0)),
            pl.BlockSpec((r3, 1), lambda i: (0, 0)),
            pl.BlockSpec((r3, 1), lambda i: (0, 0)),
            pl.BlockSpec((h, 1), lambda i: (0, 0)),
        ],
        out_specs=pl.BlockSpec((h, 1), lambda i: (0, 0)),
        out_shape=jax.ShapeDtypeStruct((h, 1), jnp.float32),
        scratch_shapes=[
            pltpu.VMEM((r3, 1), jnp.float32),
            pltpu.VMEM((r3, 1), jnp.float32),
        ],
    )(wih, whh, zcol, hcol, bihc, bhhc, h0c)


def _xw_body(x_ref, wnt_ref, deg_ref, yw_ref):
    dinv = lax.rsqrt(deg_ref[0, :] + deg_ref[1, :] + 1.0)
    yw_ref[...] = dinv[:, None] * jnp.dot(x_ref[...], wnt_ref[...],
                                          preferred_element_type=jnp.float32)


def _xw(xp, wnt, deg, bn_rows):
    d = xp.shape[1]
    h = wnt.shape[1]
    return pl.pallas_call(
        _xw_body,
        grid=(NPAD // bn_rows,),
        in_specs=[
            pl.BlockSpec((bn_rows, d), lambda i: (i, 0)),
            pl.BlockSpec((d, h), lambda i: (0, 0)),
            pl.BlockSpec((2, bn_rows), lambda i: (0, i)),
        ],
        out_specs=pl.BlockSpec((bn_rows, h), lambda i: (i, 0)),
        out_shape=jax.ShapeDtypeStruct((NPAD, h), jnp.float32),
    )(xp, wnt, deg)


def _post_body(agg_ref, yw_ref, deg_ref, bn_ref, g_ref, b_ref, out_ref, *,
               do_ln):
    dinv = lax.rsqrt(deg_ref[0, :] + deg_ref[1, :] + 1.0)
    o = dinv[:, None] * (agg_ref[0] + agg_ref[1] + yw_ref[...]) \
        + bn_ref[0, :][None, :]
    if do_ln:
        mu = jnp.mean(o, axis=-1, keepdims=True)
        var = jnp.mean((o - mu) ** 2, axis=-1, keepdims=True)
        o = (o - mu) * lax.rsqrt(var + 1e-5) * g_ref[0, :][None, :] \
            + b_ref[0, :][None, :]
        o = jnp.maximum(o, 0.0)
    out_ref[...] = o


def _post(agg, yw, deg, bn2, g2, b2, do_ln, bn_rows):
    h = yw.shape[1]
    return pl.pallas_call(
        functools.partial(_post_body, do_ln=do_ln),
        grid=(NPAD // bn_rows,),
        in_specs=[
            pl.BlockSpec((2, bn_rows, h), lambda i: (0, i, 0)),
            pl.BlockSpec((bn_rows, h), lambda i: (i, 0)),
            pl.BlockSpec((2, bn_rows), lambda i: (0, i)),
            pl.BlockSpec((1, h), lambda i: (0, 0)),
            pl.BlockSpec((1, h), lambda i: (0, 0)),
            pl.BlockSpec((1, h), lambda i: (0, 0)),
        ],
        out_specs=pl.BlockSpec((bn_rows, h), lambda i: (i, 0)),
        out_shape=jax.ShapeDtypeStruct((NPAD, h), jnp.float32),
    )(agg, yw, deg, bn2, g2, b2)


def kernel(x, edge_index, W0, b0, W1, b1, g0wih, g0whh, g0bih, g0bhh,
           g1wih, g1whh, g1bih, g1bhh, p0, p1, ln_g, ln_b):
    src = edge_index[0]
    dst = edge_index[1]
    padi = jnp.full((EPAD - E,), N, jnp.int32)
    src3 = jnp.concatenate([src, padi]).reshape(NC, NS, NCH, CH)
    dst3 = jnp.concatenate([dst, padi]).reshape(NC, NS, NCH, CH)
    zeros1 = jnp.zeros((NPAD,), jnp.float32)
    zeros2 = jnp.zeros((NPAD, HID), jnp.float32)

    deg = _sc_deg(dst3, zeros1)
    xpad = jnp.concatenate(
        [x, jnp.zeros((NPAD - N, D), jnp.float32)], axis=0)

    Z0 = _front(xpad, p0.reshape(1, D), N, TK, D)
    hid0 = jnp.concatenate([W0.reshape(-1), b0])
    v0 = _gru(g0wih, g0whh, Z0.reshape(-1)[:, None], hid0[:, None],
              g0bih[:, None], g0bhh[:, None], hid0[:, None],
              1032)[:, 0]
    wnt0 = v0[:HID * D].reshape(HID, D).T
    bn0 = v0[HID * D:].reshape(1, HID)
    yw0 = _xw(xpad, wnt0, deg, 2048)
    agg0 = _sc_agg(yw0, src3, dst3, zeros2)
    h1 = _post(agg0, yw0, deg, bn0, ln_g.reshape(1, HID),
               ln_b.reshape(1, HID), True, 2048)

    Z1 = _front(h1, p1.reshape(1, HID), N, TK, HID)
    hid1 = jnp.concatenate([W1.reshape(-1), b1])
    v1 = _gru(g1wih, g1whh, Z1.reshape(-1)[:, None], hid1[:, None],
              g1bih[:, None], g1bhh[:, None], hid1[:, None],
              3 * H1)[:, 0]
    wnt1 = v1[:OUT * HID].reshape(OUT, HID).T
    bn1 = v1[OUT * HID:].reshape(1, OUT)
    yw1 = _xw(h1, wnt1, deg, 2048)
    agg1 = _sc_agg(yw1, src3, dst3, zeros2)
    h2 = _post(agg1, yw1, deg, bn1, ln_g.reshape(1, HID),
               ln_b.reshape(1, HID), False, 2048)

    return h2[:N]
